# baseline (speedup 1.0000x reference)
import jax
import jax.numpy as jnp
from jax import lax
from jax.experimental import pallas as pl
from jax.experimental.pallas import tpu as pltpu
from jax.experimental.pallas import tpu_sc as plsc

N_NODES = 10000
N_EDGES = 320000
D = 128

NC = 2
NS = 16
NW = NC * NS

CHUNK = 64
CPT = 160
SEG = 16
NSEG = CPT // SEG
E_PAD = CHUNK * CPT * NW
N_CHUNK_ROWS = E_PAD // CHUNK
N_PAD = 10240
NPR = N_PAD // 128
ROWS_PER_TILE = N_PAD // NS


def _agg_body(x_hbm, ed_hbm, out_hbm, cntout_hbm,
              seg, rows0, rows1, rows2, rows3, cnt_v, acc,
              gsem0, gsem1, gsem2, gsem3, ssem0, ssem1, ssem2, ssem3, isem):
    c = lax.axis_index("c")
    s = lax.axis_index("s")
    wid = s * NC + c
    base = wid * CPT

    rows = (rows0, rows1, rows2, rows3)
    gsem = (gsem0, gsem1, gsem2, gsem3)
    ssem = (ssem0, ssem1, ssem2, ssem3)

    def zero_rows0(i, carry):
        for cc in range(D // 16):
            rows0[i, pl.ds(cc * 16, 16)] = jnp.zeros((16,), jnp.float32)
        return carry
    lax.fori_loop(0, CHUNK, zero_rows0, 0)

    def zero_cnt(i, carry):
        for cc in range(D // 16):
            cnt_v[i, pl.ds(cc * 16, 16)] = jnp.zeros((16,), jnp.float32)
        return carry
    lax.fori_loop(0, NPR, zero_cnt, 0)

    row0 = s * ROWS_PER_TILE

    def zero_acc(i, carry):
        pltpu.sync_copy(rows0, acc.at[pl.ds(row0 + i * CHUNK, CHUNK)])
        return carry
    lax.fori_loop(0, ROWS_PER_TILE // CHUNK, zero_acc, 0)

    plsc.subcore_barrier()

    pltpu.sync_copy(ed_hbm.at[pl.ds(base, SEG)], seg.at[0])

    def src_idx(q):
        return seg.at[(q // SEG) % 2, lax.rem(q, SEG), 0]

    def dst_idx(q):
        return seg.at[(q // SEG) % 2, lax.rem(q, SEG), 1]

    def issue(q, b):
        pltpu.async_copy(x_hbm.at[src_idx(q)], rows[b], gsem[b])

    def drain_g(b):
        pltpu.make_async_copy(x_hbm.at[pl.ds(0, CHUNK)], rows[b], gsem[b]).wait()

    def scatter(q, b):
        pltpu.async_copy(rows[b], acc.at[dst_idx(q)], ssem[b], add=True)

    def drain_s(b):
        pltpu.make_async_copy(rows[b], acc.at[pl.ds(0, CHUNK)], ssem[b]).wait()

    ones16 = jnp.ones((16,), jnp.float32)

    def count(q):
        a = (q // SEG) % 2
        r = lax.rem(q, SEG)
        for cc in range(CHUNK // 16):
            idx = seg[a, r, 1, pl.ds(cc * 16, 16)]
            plsc.addupdate_scatter(
                cnt_v, [lax.shift_right_logical(idx, 7),
                        lax.bitwise_and(idx, 127)], ones16)

    def step(q, b, first):
        nb = (b + 2) % 4
        k1 = q // SEG + 1

        if first:
            @pl.when(q >= 2)
            def _():
                drain_s(nb)
        else:
            drain_s(nb)

        @pl.when(jnp.logical_and(lax.rem(q, SEG) == 1, k1 < NSEG))
        def _():
            pltpu.async_copy(ed_hbm.at[pl.ds(base + k1 * SEG, SEG)],
                             seg.at[lax.rem(k1, 2)], isem)

        @pl.when(jnp.logical_and(lax.rem(q, SEG) == SEG - 2, k1 < NSEG))
        def _():
            pltpu.make_async_copy(ed_hbm.at[pl.ds(0, SEG)], seg.at[0],
                                  isem).wait()

        @pl.when(q + 2 < CPT)
        def _():
            issue(q + 2, nb)

        drain_g(b)
        scatter(q, b)
        count(q)

    issue(0, 0)
    issue(1, 1)

    def group(g, carry):
        step(g * 4, 0, True)
        step(g * 4 + 1, 1, True)
        step(g * 4 + 2, 2, False)
        step(g * 4 + 3, 3, False)
        return carry
    lax.fori_loop(0, CPT // 4, group, 0)

    drain_s(2)
    drain_s(3)

    plsc.subcore_barrier()

    pltpu.sync_copy(acc.at[pl.ds(row0, ROWS_PER_TILE)],
                    out_hbm.at[c, pl.ds(row0, ROWS_PER_TILE)])
    pltpu.sync_copy(cnt_v, cntout_hbm.at[wid])


_agg = pl.kernel(
    _agg_body,
    out_type=[
        jax.ShapeDtypeStruct((NC, N_PAD, D), jnp.float32),
        jax.ShapeDtypeStruct((NW, NPR, 128), jnp.float32),
    ],
    compiler_params=pltpu.CompilerParams(
        use_tc_tiling_on_sc=False, needs_layout_passes=False),
    mesh=plsc.VectorSubcoreMesh(core_axis_name="c", subcore_axis_name="s"),
    scratch_types=[
        pltpu.VMEM((2, SEG, 2, CHUNK), jnp.int32),
        pltpu.VMEM((CHUNK, D), jnp.float32),
        pltpu.VMEM((CHUNK, D), jnp.float32),
        pltpu.VMEM((CHUNK, D), jnp.float32),
        pltpu.VMEM((CHUNK, D), jnp.float32),
        pltpu.VMEM((NPR, 128), jnp.float32),
        pltpu.VMEM_SHARED((N_PAD, D), jnp.float32),
        pltpu.SemaphoreType.DMA,
        pltpu.SemaphoreType.DMA,
        pltpu.SemaphoreType.DMA,
        pltpu.SemaphoreType.DMA,
        pltpu.SemaphoreType.DMA,
        pltpu.SemaphoreType.DMA,
        pltpu.SemaphoreType.DMA,
        pltpu.SemaphoreType.DMA,
        pltpu.SemaphoreType.DMA,
    ],
)


_BLK = 2048


def _dense_body(p0, p1, cn, xr, wl, bl, wr, o):
    rcp = 1.0 / jnp.maximum(cn[...], 1.0)
    mean = (p0[...] + p1[...]) * rcp
    acc = jnp.dot(mean, wl[...], preferred_element_type=jnp.float32)
    acc = acc + jnp.dot(xr[...], wr[...], preferred_element_type=jnp.float32)
    o[...] = jnp.maximum(acc + bl[...], 0.0)


def _dense(sums, cnts, x, Wl, bl, Wr):
    return pl.pallas_call(
        _dense_body,
        grid=(N_PAD // _BLK,),
        in_specs=[
            pl.BlockSpec((None, _BLK, D), lambda i: (0, i, 0)),
            pl.BlockSpec((None, _BLK, D), lambda i: (1, i, 0)),
            pl.BlockSpec((_BLK, 1), lambda i: (i, 0)),
            pl.BlockSpec((_BLK, D), lambda i: (i, 0)),
            pl.BlockSpec((D, D), lambda i: (0, 0)),
            pl.BlockSpec((1, D), lambda i: (0, 0)),
            pl.BlockSpec((D, D), lambda i: (0, 0)),
        ],
        out_specs=pl.BlockSpec((_BLK, D), lambda i: (i, 0)),
        out_shape=jax.ShapeDtypeStruct((N_NODES, D), jnp.float32),
    )(sums, sums, cnts, x, Wl.T, bl.reshape(1, D), Wr.T)


def kernel(x, edge_index, W1l, b1l, W1r, W2l, b2l, W2r):
    src = edge_index[0].astype(jnp.int32)
    dst = edge_index[1].astype(jnp.int32)
    pad = E_PAD - N_EDGES
    pad_src = (jnp.arange(pad, dtype=jnp.int32) * 61) % N_NODES
    pad_dst = N_NODES + (jnp.arange(pad, dtype=jnp.int32) % (N_PAD - N_NODES))
    src2d = jnp.concatenate([src, pad_src]).reshape(N_CHUNK_ROWS, CHUNK)
    dst2d = jnp.concatenate([dst, pad_dst]).reshape(N_CHUNK_ROWS, CHUNK)
    ed2d = jnp.stack([src2d, dst2d], axis=1)

    sums1, cnts1 = _agg(x, ed2d)
    cnt_col1 = jnp.sum(cnts1, axis=0).reshape(N_PAD, 1)
    hid = _dense(sums1, cnt_col1, x, W1l, b1l, W1r)
    sums2, cnts2 = _agg(hid, ed2d)
    cnt_col2 = jnp.sum(cnts2, axis=0).reshape(N_PAD, 1)
    out = _dense(sums2, cnt_col2, hid, W2l, b2l, W2r)
    return out


# speedup vs baseline: 1.0107x; 1.0107x over previous
import jax
import jax.numpy as jnp
from jax import lax
from jax.experimental import pallas as pl
from jax.experimental.pallas import tpu as pltpu
from jax.experimental.pallas import tpu_sc as plsc

N_NODES = 10000
N_EDGES = 320000
D = 128

NC = 2
NS = 16
NW = NC * NS

CHUNK = 64
CPT = 160
SEG = 32
NSEG = CPT // SEG
E_PAD = CHUNK * CPT * NW
N_CHUNK_ROWS = E_PAD // CHUNK
N_PAD = 10240
NPR = N_PAD // 128
ROWS_PER_TILE = N_PAD // NS


def _agg_body(x_hbm, ed_hbm, out_hbm, cntout_hbm,
              seg, rows0, rows1, rows2, cnt_v, acc,
              gsem0, gsem1, gsem2, ssem0, ssem1, ssem2, isem):
    c = lax.axis_index("c")
    s = lax.axis_index("s")
    wid = s * NC + c
    base = wid * CPT

    rows = (rows0, rows1, rows2)
    gsem = (gsem0, gsem1, gsem2)
    ssem = (ssem0, ssem1, ssem2)

    def zero_rows0(i, carry):
        for cc in range(D // 16):
            rows0[i, pl.ds(cc * 16, 16)] = jnp.zeros((16,), jnp.float32)
        return carry
    lax.fori_loop(0, CHUNK, zero_rows0, 0)

    def zero_cnt(i, carry):
        for cc in range(D // 16):
            cnt_v[i, pl.ds(cc * 16, 16)] = jnp.zeros((16,), jnp.float32)
        return carry
    lax.fori_loop(0, NPR, zero_cnt, 0)

    row0 = s * ROWS_PER_TILE

    def zero_acc(i, carry):
        pltpu.sync_copy(rows0, acc.at[pl.ds(row0 + i * CHUNK, CHUNK)])
        return carry
    lax.fori_loop(0, ROWS_PER_TILE // CHUNK, zero_acc, 0)

    plsc.subcore_barrier()

    pltpu.sync_copy(ed_hbm.at[pl.ds(base, SEG)], seg.at[0])

    def src_idx(q):
        return seg.at[(q // SEG) % 2, lax.rem(q, SEG), 0]

    def dst_idx(q):
        return seg.at[(q // SEG) % 2, lax.rem(q, SEG), 1]

    def issue(q, b):
        pltpu.async_copy(x_hbm.at[src_idx(q)], rows[b], gsem[b])

    def drain_g(b):
        pltpu.make_async_copy(x_hbm.at[pl.ds(0, CHUNK)], rows[b], gsem[b]).wait()

    def scatter(q, b):
        pltpu.async_copy(rows[b], acc.at[dst_idx(q)], ssem[b], add=True)

    def drain_s(b):
        pltpu.make_async_copy(rows[b], acc.at[pl.ds(0, CHUNK)], ssem[b]).wait()

    ones16 = jnp.ones((16,), jnp.float32)

    def count(q):
        a = (q // SEG) % 2
        r = lax.rem(q, SEG)
        for cc in range(CHUNK // 16):
            idx = seg[a, r, 1, pl.ds(cc * 16, 16)]
            plsc.addupdate_scatter(
                cnt_v, [lax.shift_right_logical(idx, 7),
                        lax.bitwise_and(idx, 127)], ones16)

    def step(q, b, first):
        nb = (b + 2) % 3
        k1 = q // SEG + 1

        if first:
            @pl.when(q >= 1)
            def _():
                drain_s(nb)
        else:
            drain_s(nb)

        @pl.when(jnp.logical_and(lax.rem(q, SEG) == 0, k1 < NSEG))
        def _():
            pltpu.async_copy(ed_hbm.at[pl.ds(base + k1 * SEG, SEG)],
                             seg.at[lax.rem(k1, 2)], isem)

        @pl.when(jnp.logical_and(lax.rem(q, SEG) == SEG - 2, k1 < NSEG))
        def _():
            pltpu.make_async_copy(ed_hbm.at[pl.ds(0, SEG)], seg.at[0],
                                  isem).wait()

        @pl.when(q + 2 < CPT)
        def _():
            issue(q + 2, nb)

        drain_g(b)
        scatter(q, b)
        count(q)

    issue(0, 0)
    issue(1, 1)

    def group(g, carry):
        step(g * 3, 0, True)
        step(g * 3 + 1, 1, False)
        step(g * 3 + 2, 2, False)
        return carry
    lax.fori_loop(0, (CPT - 1) // 3, group, 0)

    qt = CPT - 1
    drain_g(0)
    scatter(qt, 0)
    count(qt)
    drain_s(2)
    drain_s(0)

    plsc.subcore_barrier()

    pltpu.sync_copy(acc.at[pl.ds(row0, ROWS_PER_TILE)],
                    out_hbm.at[c, pl.ds(row0, ROWS_PER_TILE)])
    pltpu.sync_copy(cnt_v, cntout_hbm.at[wid])


_agg = pl.kernel(
    _agg_body,
    out_type=[
        jax.ShapeDtypeStruct((NC, N_PAD, D), jnp.float32),
        jax.ShapeDtypeStruct((NW, NPR, 128), jnp.float32),
    ],
    compiler_params=pltpu.CompilerParams(
        use_tc_tiling_on_sc=False, needs_layout_passes=False),
    mesh=plsc.VectorSubcoreMesh(core_axis_name="c", subcore_axis_name="s"),
    scratch_types=[
        pltpu.VMEM((2, SEG, 2, CHUNK), jnp.int32),
        pltpu.VMEM((CHUNK, D), jnp.float32),
        pltpu.VMEM((CHUNK, D), jnp.float32),
        pltpu.VMEM((CHUNK, D), jnp.float32),
        pltpu.VMEM((NPR, 128), jnp.float32),
        pltpu.VMEM_SHARED((N_PAD, D), jnp.float32),
        pltpu.SemaphoreType.DMA,
        pltpu.SemaphoreType.DMA,
        pltpu.SemaphoreType.DMA,
        pltpu.SemaphoreType.DMA,
        pltpu.SemaphoreType.DMA,
        pltpu.SemaphoreType.DMA,
        pltpu.SemaphoreType.DMA,
    ],
)


_BLK = 2048


def _dense_body(p0, p1, cn, xr, wl, bl, wr, o):
    rcp = 1.0 / jnp.maximum(cn[...], 1.0)
    mean = (p0[...] + p1[...]) * rcp
    acc = jnp.dot(mean, wl[...], preferred_element_type=jnp.float32)
    acc = acc + jnp.dot(xr[...], wr[...], preferred_element_type=jnp.float32)
    o[...] = jnp.maximum(acc + bl[...], 0.0)


def _dense(sums, cnts, x, Wl, bl, Wr):
    return pl.pallas_call(
        _dense_body,
        grid=(N_PAD // _BLK,),
        in_specs=[
            pl.BlockSpec((None, _BLK, D), lambda i: (0, i, 0)),
            pl.BlockSpec((None, _BLK, D), lambda i: (1, i, 0)),
            pl.BlockSpec((_BLK, 1), lambda i: (i, 0)),
            pl.BlockSpec((_BLK, D), lambda i: (i, 0)),
            pl.BlockSpec((D, D), lambda i: (0, 0)),
            pl.BlockSpec((1, D), lambda i: (0, 0)),
            pl.BlockSpec((D, D), lambda i: (0, 0)),
        ],
        out_specs=pl.BlockSpec((_BLK, D), lambda i: (i, 0)),
        out_shape=jax.ShapeDtypeStruct((N_NODES, D), jnp.float32),
    )(sums, sums, cnts, x, Wl.T, bl.reshape(1, D), Wr.T)


def kernel(x, edge_index, W1l, b1l, W1r, W2l, b2l, W2r):
    src = edge_index[0].astype(jnp.int32)
    dst = edge_index[1].astype(jnp.int32)
    pad = E_PAD - N_EDGES
    pad_src = (jnp.arange(pad, dtype=jnp.int32) * 61) % N_NODES
    pad_dst = N_NODES + (jnp.arange(pad, dtype=jnp.int32) % (N_PAD - N_NODES))
    src2d = jnp.concatenate([src, pad_src]).reshape(N_CHUNK_ROWS, CHUNK)
    dst2d = jnp.concatenate([dst, pad_dst]).reshape(N_CHUNK_ROWS, CHUNK)
    ed2d = jnp.stack([src2d, dst2d], axis=1)

    sums1, cnts1 = _agg(x, ed2d)
    cnt_col1 = jnp.sum(cnts1, axis=0).reshape(N_PAD, 1)
    hid = _dense(sums1, cnt_col1, x, W1l, b1l, W1r)
    sums2, cnts2 = _agg(hid, ed2d)
    cnt_col2 = jnp.sum(cnts2, axis=0).reshape(N_PAD, 1)
    out = _dense(sums2, cnt_col2, hid, W2l, b2l, W2r)
    return out
